# baseline (speedup 1.0000x reference)
import functools

import jax
import jax.numpy as jnp
from jax import lax
from jax.experimental import pallas as pl
from jax.experimental.pallas import tpu as pltpu
from jax.experimental.pallas import tpu_sc as plsc
from jax.experimental import layout as jlayout

NC = 2
NS = 16
NW = NC * NS

CL = 2


def _make_kernel(B, L, V, D):
  bpw = B // NW
  lt = L // 8
  dt = D // 8
  n_chunks = L // CL
  crows = CL * bpw

  mesh = plsc.VectorSubcoreMesh(
      core_axis_name="c", subcore_axis_name="s", num_cores=NC,
      num_subcores=NS)

  @functools.partial(
      pl.kernel,
      out_type=jax.ShapeDtypeStruct((L, dt, NW * 8 * bpw), jnp.float32),
      mesh=mesh,
      scratch_types=[
          pltpu.VMEM((L, bpw), jnp.int32),
          pltpu.VMEM((D, L), jnp.float32),
          pltpu.VMEM((2, crows, D), jnp.float32),
          pltpu.VMEM((2, CL, dt, 8 * bpw), jnp.float32),
          pltpu.SemaphoreType.DMA((2,)),
          pltpu.SemaphoreType.DMA((2,)),
      ],
      compiler_params=pltpu.CompilerParams(use_tc_tiling_on_sc=False,
                                           needs_layout_passes=False),
  )
  def k(seqf_hbm, tok_hbm, post_hbm, out_hbm, idx_v, pos_v, buf, obuf,
        gsem, ssem):
    cid = lax.axis_index("c")
    sid = lax.axis_index("s")
    wid = sid * NC + cid
    pltpu.sync_copy(post_hbm, pos_v)
    for i in range(lt):
      pltpu.sync_copy(seqf_hbm.at[pl.ds(i * NW * 8 + wid * 8, 8)],
                      idx_v.at[pl.ds(i * 8, 8)])

    def fire(c, bb):
      for cl in range(CL):
        pltpu.async_copy(tok_hbm.at[idx_v.at[c * CL + cl]],
                         buf.at[bb, pl.ds(cl * bpw, bpw)], gsem.at[bb])

    def drain_gather(bb):
      pltpu.make_async_copy(tok_hbm.at[pl.ds(0, crows)], buf.at[bb],
                            gsem.at[bb]).wait()

    def drain_store(bb):
      pltpu.make_async_copy(
          obuf.at[bb],
          out_hbm.at[pl.ds(0, CL), pl.ds(0, dt), pl.ds(0, 8 * bpw)],
          ssem.at[bb]).wait()

    fire(0, 0)

    def body(c, carry):
      bb = lax.rem(c, 2)
      nb = 1 - bb

      @pl.when(c > 0)
      def _():
        drain_store(nb)

      @pl.when(c < n_chunks - 1)
      def _():
        fire(c + 1, nb)

      drain_gather(bb)
      for cl in range(CL):
        l = c * CL + cl

        def dbody(dd, carry2, cl=cl, l=l):
          iota = jnp.arange(16, dtype=jnp.int32)
          p = plsc.load_gather(
              pos_v, [jnp.full((16,), dd, jnp.int32),
                      jnp.full((16,), l, jnp.int32)])
          t = dd // 8
          r = dd % 8
          fdd = jnp.full((16,), dd, jnp.int32)
          for b8 in range(bpw // 16):
            v = plsc.load_gather(
                buf.at[bb], [cl * bpw + b8 * 16 + iota, fdd])
            obuf[bb, cl, t, pl.ds(r * bpw + b8 * 16, 16)] = v + p
          return carry2

        lax.fori_loop(0, D, dbody, 0)

      pltpu.async_copy(
          obuf.at[bb],
          out_hbm.at[pl.ds(c * CL, CL), pl.ds(0, dt),
                     pl.ds(wid * 8 * bpw, 8 * bpw)],
          ssem.at[bb])
      return carry

    lax.fori_loop(0, n_chunks, body, 0)
    drain_store((n_chunks - 1) % 2)

  return k


def kernel(seq, token_table, pos_table):
  B, L = seq.shape
  V, D = token_table.shape
  seqf = (seq.astype(jnp.int32)
          .reshape(NW, B // NW, L // 8, 8).transpose(2, 0, 3, 1)
          .reshape((L // 8) * NW * 8, B // NW))
  pos_t = pos_table.T
  k = _make_kernel(B, L, V, D)
  out3 = k(seqf, token_table, pos_t)
  out = (out3.reshape(L, D // 8, NW, 8, B // NW)
         .transpose(2, 4, 0, 1, 3).reshape(B, L, D))
  return jlayout.with_layout_constraint(
      out, jlayout.Layout(major_to_minor=(1, 2, 0), tiling=((8, 128),)))


# speedup vs baseline: 1.7737x; 1.7737x over previous
import functools

import jax
import jax.numpy as jnp
from jax import lax
from jax.experimental import pallas as pl
from jax.experimental.pallas import tpu as pltpu
from jax.experimental.pallas import tpu_sc as plsc
from jax.experimental import layout as jlayout

NC = 2
NS = 16
NW = NC * NS

CL = 2


def _make_kernel(B, L, V, D):
  bpw = B // NW
  lt = L // 8
  dt = D // 8
  n_chunks = L // CL
  crows = CL * bpw

  mesh = plsc.VectorSubcoreMesh(
      core_axis_name="c", subcore_axis_name="s", num_cores=NC,
      num_subcores=NS)

  @functools.partial(
      pl.kernel,
      out_type=jax.ShapeDtypeStruct((L, dt, NW * 8 * bpw), jnp.float32),
      mesh=mesh,
      scratch_types=[
          pltpu.VMEM((L, bpw), jnp.int32),
          pltpu.VMEM((D, L), jnp.float32),
          pltpu.VMEM((2, crows, D), jnp.float32),
          pltpu.VMEM((2, CL, dt, 8 * bpw), jnp.float32),
          pltpu.SemaphoreType.DMA((2,)),
          pltpu.SemaphoreType.DMA((2,)),
      ],
      compiler_params=pltpu.CompilerParams(use_tc_tiling_on_sc=False,
                                           needs_layout_passes=False),
  )
  def k(seqf_hbm, tok_hbm, post_hbm, out_hbm, idx_v, pos_v, buf, obuf,
        gsem, ssem):
    cid = lax.axis_index("c")
    sid = lax.axis_index("s")
    wid = sid * NC + cid
    pltpu.sync_copy(post_hbm, pos_v)
    for i in range(lt):
      pltpu.sync_copy(seqf_hbm.at[pl.ds(i * NW * 8 + wid * 8, 8)],
                      idx_v.at[pl.ds(i * 8, 8)])

    def fire(c, bb):
      for cl in range(CL):
        pltpu.async_copy(tok_hbm.at[idx_v.at[c * CL + cl]],
                         buf.at[bb, pl.ds(cl * bpw, bpw)], gsem.at[bb])

    def drain_gather(bb):
      pltpu.make_async_copy(tok_hbm.at[pl.ds(0, crows)], buf.at[bb],
                            gsem.at[bb]).wait()

    def drain_store(bb):
      pltpu.make_async_copy(
          obuf.at[bb],
          out_hbm.at[pl.ds(0, CL), pl.ds(0, dt), pl.ds(0, 8 * bpw)],
          ssem.at[bb]).wait()

    fire(0, 0)

    def body(c, carry):
      bb = lax.rem(c, 2)
      nb = 1 - bb

      @pl.when(c > 0)
      def _():
        drain_store(nb)

      @pl.when(c < n_chunks - 1)
      def _():
        fire(c + 1, nb)

      drain_gather(bb)
      for cl in range(CL):
        l = c * CL + cl

        @plsc.parallel_loop(0, D, 1, unroll=4)
        def _(dd, cl=cl, l=l):
          iota = jnp.arange(16, dtype=jnp.int32)
          p = plsc.load_gather(
              pos_v, [jnp.full((16,), dd, jnp.int32),
                      jnp.full((16,), l, jnp.int32)])
          t = dd // 8
          r = dd % 8
          fdd = jnp.full((16,), dd, jnp.int32)
          for b8 in range(bpw // 16):
            v = plsc.load_gather(
                buf.at[bb], [cl * bpw + b8 * 16 + iota, fdd])
            obuf[bb, cl, t, pl.ds(r * bpw + b8 * 16, 16)] = v + p

      pltpu.async_copy(
          obuf.at[bb],
          out_hbm.at[pl.ds(c * CL, CL), pl.ds(0, dt),
                     pl.ds(wid * 8 * bpw, 8 * bpw)],
          ssem.at[bb])
      return carry

    lax.fori_loop(0, n_chunks, body, 0)
    drain_store((n_chunks - 1) % 2)

  return k


def kernel(seq, token_table, pos_table):
  B, L = seq.shape
  V, D = token_table.shape
  seqf = (seq.astype(jnp.int32)
          .reshape(NW, B // NW, L // 8, 8).transpose(2, 0, 3, 1)
          .reshape((L // 8) * NW * 8, B // NW))
  pos_t = pos_table.T
  k = _make_kernel(B, L, V, D)
  out3 = k(seqf, token_table, pos_t)
  out = (out3.reshape(L, D // 8, NW, 8, B // NW)
         .transpose(2, 4, 0, 1, 3).reshape(B, L, D))
  return jlayout.with_layout_constraint(
      out, jlayout.Layout(major_to_minor=(1, 2, 0), tiling=((8, 128),)))


# speedup vs baseline: 3.6182x; 2.0399x over previous
import functools

import jax
import jax.numpy as jnp
from jax import lax
from jax.experimental import pallas as pl
from jax.experimental.pallas import tpu as pltpu
from jax.experimental.pallas import tpu_sc as plsc

NC = 2
NS = 16
NW = NC * NS

G = 100


def _make_kernel(B, L, V, D):
  chunk_seq = 2
  N = B * L
  rows_per_worker = N // NW
  chunk_rows = chunk_seq * L
  n_chunks = rows_per_worker // chunk_rows
  subg = chunk_rows // G
  idx_rows = rows_per_worker // G
  nseg = D // 16

  mesh = plsc.VectorSubcoreMesh(
      core_axis_name="c", subcore_axis_name="s", num_cores=NC,
      num_subcores=NS)

  @functools.partial(
      pl.kernel,
      out_type=jax.ShapeDtypeStruct((N, 2 * D), jnp.float32),
      mesh=mesh,
      scratch_types=[
          pltpu.VMEM((idx_rows, G), jnp.int32),
          pltpu.VMEM((2, chunk_rows, D), jnp.float32),
          pltpu.VMEM((L, D), jnp.float32),
          pltpu.SemaphoreType.DMA((2,)),
          pltpu.SemaphoreType.DMA((2,)),
      ],
      compiler_params=pltpu.CompilerParams(use_tc_tiling_on_sc=False),
  )
  def k(idx_hbm, tok_hbm, pos_hbm, out_hbm, idx_v, rows_v, pos_v, gsem, ssem):
    cid = lax.axis_index("c")
    sid = lax.axis_index("s")
    wid = sid * NC + cid
    base_row = wid * rows_per_worker
    pltpu.sync_copy(pos_hbm, pos_v)
    pltpu.sync_copy(idx_hbm.at[pl.ds(wid * idx_rows, idx_rows)], idx_v)

    def fire(c, buf):
      for j in range(subg):
        pltpu.async_copy(tok_hbm.at[idx_v.at[c * subg + j]],
                         rows_v.at[buf, pl.ds(j * G, G)], gsem.at[buf])

    def drain_gather(buf):
      pltpu.make_async_copy(tok_hbm.at[pl.ds(0, chunk_rows)],
                            rows_v.at[buf], gsem.at[buf]).wait()

    def drain_store(buf):
      pltpu.make_async_copy(rows_v.at[buf],
                            out_hbm.at[pl.ds(0, chunk_rows), pl.ds(0, D)],
                            ssem.at[buf]).wait()

    fire(0, 0)

    def body(c, carry):
      buf = lax.rem(c, 2)
      nbuf = 1 - buf

      @pl.when(c > 0)
      def _():
        drain_store(nbuf)

      @pl.when(c < n_chunks - 1)
      def _():
        fire(c + 1, nbuf)

      drain_gather(buf)

      @plsc.parallel_loop(0, L, 1, unroll=4)
      def _(l):
        for seg in range(nseg):
          p = pos_v[l, pl.ds(seg * 16, 16)]
          for s in range(chunk_seq):
            plsc.addupdate(rows_v.at[buf, s * L + l, pl.ds(seg * 16, 16)], p)

      pltpu.async_copy(
          rows_v.at[buf],
          out_hbm.at[pl.ds(base_row + c * chunk_rows, chunk_rows), pl.ds(0, D)],
          ssem.at[buf])
      return carry

    lax.fori_loop(0, n_chunks, body, 0)
    drain_store((n_chunks - 1) % 2)

  return k


def kernel(seq, token_table, pos_table):
  B, L = seq.shape
  V, D = token_table.shape
  N = B * L
  idx = seq.reshape(N // G, G).astype(jnp.int32)
  k = _make_kernel(B, L, V, D)
  out = k(idx, token_table, pos_table)
  return out[:, :D].reshape(B, L, D)


# speedup vs baseline: 3.6234x; 1.0014x over previous
import functools

import jax
import jax.numpy as jnp
from jax import lax
from jax.experimental import pallas as pl
from jax.experimental.pallas import tpu as pltpu
from jax.experimental.pallas import tpu_sc as plsc

NC = 2
NS = 16
NW = NC * NS

G = 100


def _make_kernel(B, L, V, D):
  chunk_seq = 2
  N = B * L
  rows_per_worker = N // NW
  chunk_rows = chunk_seq * L
  n_chunks = rows_per_worker // chunk_rows
  subg = chunk_rows // G
  idx_rows = rows_per_worker // G
  nseg = D // 16

  mesh = plsc.VectorSubcoreMesh(
      core_axis_name="c", subcore_axis_name="s", num_cores=NC,
      num_subcores=NS)

  @functools.partial(
      pl.kernel,
      out_type=jax.ShapeDtypeStruct((N, 2 * D), jnp.float32),
      mesh=mesh,
      scratch_types=[
          pltpu.VMEM((idx_rows, G), jnp.int32),
          pltpu.VMEM((2, chunk_rows, D), jnp.float32),
          pltpu.VMEM((L, D), jnp.float32),
          pltpu.SemaphoreType.DMA((2,)),
          pltpu.SemaphoreType.DMA((2,)),
      ],
      compiler_params=pltpu.CompilerParams(use_tc_tiling_on_sc=False),
  )
  def k(idx_hbm, tok_hbm, pos_hbm, out_hbm, idx_v, rows_v, pos_v, gsem, ssem):
    cid = lax.axis_index("c")
    sid = lax.axis_index("s")
    wid = sid * NC + cid
    base_row = wid * rows_per_worker
    pltpu.sync_copy(pos_hbm, pos_v)
    pltpu.sync_copy(idx_hbm.at[pl.ds(wid * idx_rows, idx_rows)], idx_v)

    def fire(c, buf):
      for j in range(subg):
        pltpu.async_copy(tok_hbm.at[idx_v.at[c * subg + j]],
                         rows_v.at[buf, pl.ds(j * G, G)], gsem.at[buf])

    def drain_gather(buf):
      pltpu.make_async_copy(tok_hbm.at[pl.ds(0, chunk_rows)],
                            rows_v.at[buf], gsem.at[buf]).wait()

    def drain_store(buf):
      pltpu.make_async_copy(rows_v.at[buf],
                            out_hbm.at[pl.ds(0, chunk_rows), pl.ds(0, D)],
                            ssem.at[buf]).wait()

    fire(0, 0)

    def body(c, carry):
      buf = lax.rem(c, 2)
      nbuf = 1 - buf

      @pl.when(c > 0)
      def _():
        drain_store(nbuf)

      @pl.when(c < n_chunks - 1)
      def _():
        fire(c + 1, nbuf)

      drain_gather(buf)

      @plsc.parallel_loop(0, L, 1, unroll=8)
      def _(l):
        for seg in range(nseg):
          p = pos_v[l, pl.ds(seg * 16, 16)]
          for s in range(chunk_seq):
            plsc.addupdate(rows_v.at[buf, s * L + l, pl.ds(seg * 16, 16)], p)

      pltpu.async_copy(
          rows_v.at[buf],
          out_hbm.at[pl.ds(base_row + c * chunk_rows, chunk_rows), pl.ds(0, D)],
          ssem.at[buf])
      return carry

    lax.fori_loop(0, n_chunks, body, 0)
    drain_store((n_chunks - 1) % 2)

  return k


def kernel(seq, token_table, pos_table):
  B, L = seq.shape
  V, D = token_table.shape
  N = B * L
  idx = seq.reshape(N // G, G).astype(jnp.int32)
  k = _make_kernel(B, L, V, D)
  out = k(idx, token_table, pos_table)
  return out[:, :D].reshape(B, L, D)
